# baseline (speedup 1.0000x reference)
import functools

import jax
import jax.numpy as jnp
from jax import lax
from jax.experimental import pallas as pl
from jax.experimental.pallas import tpu as pltpu
from jax.experimental.pallas import tpu_sc as plsc

_DIM = 64
_K = 1024
_BLOCK_M = 2048
_M = 32 * 576

_NC = 2
_NS = 16
_NW = _NC * _NS
_B_PER_W = _M // _NW
_CH = 96
_NCHUNK = _B_PER_W // _CH


def _dist_argmin_body(x_ref, e2_ref, en_ref, i_ref):
    xb = x_ref[...]
    em2 = e2_ref[...]
    a = jnp.sum(xb * xb, axis=1, keepdims=True)
    prod = jax.lax.dot_general(
        xb, em2, (((1,), (1,)), ((), ())),
        preferred_element_type=jnp.float32)
    dist = (a + prod) + en_ref[...]
    m = jnp.min(dist, axis=1, keepdims=True)
    iota = jax.lax.broadcasted_iota(jnp.int32, dist.shape, 1)
    masked = jnp.where(dist == m, iota, _K)
    i_ref[...] = jnp.min(masked, axis=1).astype(jnp.int32)


def _sc_gather(table_hbm, idx_hbm, out_hbm, idx_v, rows_v, sem):
    wid = lax.axis_index("s") * _NC + lax.axis_index("c")
    base = wid * _B_PER_W
    pltpu.sync_copy(idx_hbm.at[pl.ds(base, _B_PER_W)], idx_v)
    copies = [
        pltpu.async_copy(
            table_hbm.at[idx_v.at[pl.ds(j * _CH, _CH)]],
            rows_v.at[pl.ds(j * _CH, _CH), :],
            sem)
        for j in range(_NCHUNK)
    ]
    for c in copies:
        c.wait()
    pltpu.sync_copy(rows_v, out_hbm.at[pl.ds(base, _B_PER_W)])


def kernel(x, embed):
    shape = x.shape
    flat = x.reshape(-1, shape[-1])
    en = jnp.sum((embed.T) ** 2, axis=0)[None, :]
    idx = pl.pallas_call(
        _dist_argmin_body,
        grid=(_M // _BLOCK_M,),
        in_specs=[
            pl.BlockSpec((_BLOCK_M, _DIM), lambda i: (i, 0)),
            pl.BlockSpec((_K, _DIM), lambda i: (0, 0)),
            pl.BlockSpec((1, _K), lambda i: (0, 0)),
        ],
        out_specs=pl.BlockSpec((_BLOCK_M,), lambda i: (i,)),
        out_shape=jax.ShapeDtypeStruct((_M,), jnp.int32),
        compiler_params=pltpu.CompilerParams(
            dimension_semantics=("parallel",)),
    )(flat, -2.0 * embed, en)

    mesh = plsc.VectorSubcoreMesh(core_axis_name="c", subcore_axis_name="s")
    gather = functools.partial(
        pl.kernel, mesh=mesh,
        out_type=jax.ShapeDtypeStruct((_M, 128), jnp.float32),
        scratch_types=[
            pltpu.VMEM((_B_PER_W,), jnp.int32),
            pltpu.VMEM((_B_PER_W, 128), jnp.float32),
            pltpu.SemaphoreType.DMA,
        ],
    )(_sc_gather)
    table_pad = jnp.concatenate(
        [embed, jnp.zeros((_K, 128 - _DIM), jnp.float32)], axis=1)
    quant = gather(table_pad, idx)[:, :_DIM]
    return quant.reshape(shape), idx.reshape(shape[:-1])


# speedup vs baseline: 1.1729x; 1.1729x over previous
import jax
import jax.numpy as jnp
from jax.experimental import pallas as pl
from jax.experimental.pallas import tpu as pltpu

_DIM = 64
_K = 1024
_BLOCK_M = 1024
_M = 32 * 576
_NBLK = _M // _BLOCK_M


def _vq_body(x_ref, e_ref, e2_ref, q_ref, i_ref):
    xb = x_ref[...]
    e = e_ref[...]
    em2 = e2_ref[...]
    a = jnp.sum(xb * xb, axis=1, keepdims=True)
    en = 0.25 * jnp.sum(em2 * em2, axis=1)[None, :]
    prod = jax.lax.dot_general(
        xb, em2, (((1,), (1,)), ((), ())),
        preferred_element_type=jnp.float32)
    dist = (a + prod) + en
    m = jnp.min(dist, axis=1, keepdims=True)
    iota = jax.lax.broadcasted_iota(jnp.int32, dist.shape, 1)
    masked = jnp.where(dist == m, iota, _K)
    idx = jnp.min(masked, axis=1, keepdims=True)
    i_ref[...] = idx
    onehot = (iota == idx).astype(jnp.float32)
    q_ref[...] = jax.lax.dot_general(
        onehot, e, (((1,), (0,)), ((), ())),
        preferred_element_type=jnp.float32)


def kernel(x, embed):
    shape = x.shape
    flat = x.reshape(-1, shape[-1])
    quant, idx = pl.pallas_call(
        _vq_body,
        grid=(_NBLK,),
        in_specs=[
            pl.BlockSpec((_BLOCK_M, _DIM), lambda i: (i, 0)),
            pl.BlockSpec((_K, _DIM), lambda i: (0, 0)),
            pl.BlockSpec((_K, _DIM), lambda i: (0, 0)),
        ],
        out_specs=[
            pl.BlockSpec((_BLOCK_M, _DIM), lambda i: (i, 0)),
            pl.BlockSpec((_BLOCK_M, 1), lambda i: (i, 0)),
        ],
        out_shape=[
            jax.ShapeDtypeStruct((_M, _DIM), jnp.float32),
            jax.ShapeDtypeStruct((_M, 1), jnp.int32),
        ],
        compiler_params=pltpu.CompilerParams(
            dimension_semantics=("parallel",)),
    )(flat, embed, -2.0 * embed)
    return quant.reshape(shape), idx.reshape(shape[:-1])


# speedup vs baseline: 1.2250x; 1.0444x over previous
import jax
import jax.numpy as jnp
from jax.experimental import pallas as pl
from jax.experimental.pallas import tpu as pltpu

_DIM = 64
_K = 1024
_BLOCK_M = 2048
_M = 32 * 576
_NBLK = _M // _BLOCK_M


def _vq_body(x_ref, e_ref, e2_ref, q_ref, i_ref):
    xb = x_ref[...]
    e = e_ref[...]
    em2 = e2_ref[...]
    a = jnp.sum(xb * xb, axis=1, keepdims=True)
    en = 0.25 * jnp.sum(em2 * em2, axis=1)[None, :]
    prod = jax.lax.dot_general(
        xb, em2, (((1,), (1,)), ((), ())),
        preferred_element_type=jnp.float32)
    dist = (a + prod) + en
    m = jnp.min(dist, axis=1, keepdims=True)
    iota = jax.lax.broadcasted_iota(jnp.int32, dist.shape, 1)
    masked = jnp.where(dist == m, iota, _K)
    idx = jnp.min(masked, axis=1, keepdims=True)
    i_ref[...] = idx
    onehot = (iota == idx).astype(jnp.float32)
    q_ref[...] = jax.lax.dot_general(
        onehot, e, (((1,), (0,)), ((), ())),
        preferred_element_type=jnp.float32)


def kernel(x, embed):
    shape = x.shape
    flat = x.reshape(-1, shape[-1])
    quant, idx = pl.pallas_call(
        _vq_body,
        grid=(_NBLK,),
        in_specs=[
            pl.BlockSpec((_BLOCK_M, _DIM), lambda i: (i, 0)),
            pl.BlockSpec((_K, _DIM), lambda i: (0, 0)),
            pl.BlockSpec((_K, _DIM), lambda i: (0, 0)),
        ],
        out_specs=[
            pl.BlockSpec((_BLOCK_M, _DIM), lambda i: (i, 0)),
            pl.BlockSpec((_BLOCK_M, 1), lambda i: (i, 0)),
        ],
        out_shape=[
            jax.ShapeDtypeStruct((_M, _DIM), jnp.float32),
            jax.ShapeDtypeStruct((_M, 1), jnp.int32),
        ],
        compiler_params=pltpu.CompilerParams(
            dimension_semantics=("parallel",)),
    )(flat, embed, -2.0 * embed)
    return quant.reshape(shape), idx.reshape(shape[:-1])


# speedup vs baseline: 1.2418x; 1.0138x over previous
import jax
import jax.numpy as jnp
from jax.experimental import pallas as pl
from jax.experimental.pallas import tpu as pltpu

_DIM = 64
_K = 1024
_BLOCK_M = 3072
_M = 32 * 576
_NBLK = _M // _BLOCK_M


def _vq_body(x_ref, e_ref, e2_ref, q_ref, i_ref):
    xb = x_ref[...]
    e = e_ref[...]
    em2 = e2_ref[...]
    a = jnp.sum(xb * xb, axis=1, keepdims=True)
    en = 0.25 * jnp.sum(em2 * em2, axis=1)[None, :]
    prod = jax.lax.dot_general(
        xb, em2, (((1,), (1,)), ((), ())),
        preferred_element_type=jnp.float32)
    dist = (a + prod) + en
    m = jnp.min(dist, axis=1, keepdims=True)
    iota = jax.lax.broadcasted_iota(jnp.int32, dist.shape, 1)
    masked = jnp.where(dist == m, iota, _K)
    idx = jnp.min(masked, axis=1, keepdims=True)
    i_ref[...] = idx
    onehot = (iota == idx).astype(jnp.float32)
    q_ref[...] = jax.lax.dot_general(
        onehot, e, (((1,), (0,)), ((), ())),
        preferred_element_type=jnp.float32)


def kernel(x, embed):
    shape = x.shape
    flat = x.reshape(-1, shape[-1])
    quant, idx = pl.pallas_call(
        _vq_body,
        grid=(_NBLK,),
        in_specs=[
            pl.BlockSpec((_BLOCK_M, _DIM), lambda i: (i, 0)),
            pl.BlockSpec((_K, _DIM), lambda i: (0, 0)),
            pl.BlockSpec((_K, _DIM), lambda i: (0, 0)),
        ],
        out_specs=[
            pl.BlockSpec((_BLOCK_M, _DIM), lambda i: (i, 0)),
            pl.BlockSpec((_BLOCK_M, 1), lambda i: (i, 0)),
        ],
        out_shape=[
            jax.ShapeDtypeStruct((_M, _DIM), jnp.float32),
            jax.ShapeDtypeStruct((_M, 1), jnp.int32),
        ],
        compiler_params=pltpu.CompilerParams(
            dimension_semantics=("parallel",)),
    )(flat, embed, -2.0 * embed)
    return quant.reshape(shape), idx.reshape(shape[:-1])


# speedup vs baseline: 1.2541x; 1.0099x over previous
import jax
import jax.numpy as jnp
from jax.experimental import pallas as pl
from jax.experimental.pallas import tpu as pltpu

_DIM = 64
_K = 1024
_BLOCK_M = 4608
_M = 32 * 576
_NBLK = _M // _BLOCK_M


def _vq_body(x_ref, e_ref, e2_ref, q_ref, i_ref):
    xb = x_ref[...]
    e = e_ref[...]
    em2 = e2_ref[...]
    a = jnp.sum(xb * xb, axis=1, keepdims=True)
    en = 0.25 * jnp.sum(em2 * em2, axis=1)[None, :]
    prod = jax.lax.dot_general(
        xb, em2, (((1,), (1,)), ((), ())),
        preferred_element_type=jnp.float32)
    dist = (a + prod) + en
    m = jnp.min(dist, axis=1, keepdims=True)
    iota = jax.lax.broadcasted_iota(jnp.int32, dist.shape, 1)
    masked = jnp.where(dist == m, iota, _K)
    idx = jnp.min(masked, axis=1, keepdims=True)
    i_ref[...] = idx
    onehot = (iota == idx).astype(jnp.float32)
    q_ref[...] = jax.lax.dot_general(
        onehot, e, (((1,), (0,)), ((), ())),
        preferred_element_type=jnp.float32)


def kernel(x, embed):
    shape = x.shape
    flat = x.reshape(-1, shape[-1])
    quant, idx = pl.pallas_call(
        _vq_body,
        grid=(_NBLK,),
        in_specs=[
            pl.BlockSpec((_BLOCK_M, _DIM), lambda i: (i, 0)),
            pl.BlockSpec((_K, _DIM), lambda i: (0, 0)),
            pl.BlockSpec((_K, _DIM), lambda i: (0, 0)),
        ],
        out_specs=[
            pl.BlockSpec((_BLOCK_M, _DIM), lambda i: (i, 0)),
            pl.BlockSpec((_BLOCK_M, 1), lambda i: (i, 0)),
        ],
        out_shape=[
            jax.ShapeDtypeStruct((_M, _DIM), jnp.float32),
            jax.ShapeDtypeStruct((_M, 1), jnp.int32),
        ],
        compiler_params=pltpu.CompilerParams(
            dimension_semantics=("parallel",)),
    )(flat, embed, -2.0 * embed)
    return quant.reshape(shape), idx.reshape(shape[:-1])
